# baseline (speedup 1.0000x reference)
import functools

import jax
import jax.numpy as jnp
from jax import lax
from jax.experimental import pallas as pl
from jax.experimental.pallas import tpu as pltpu
from jax.experimental.pallas import tpu_sc as plsc

_N = 10000
_E = 320000
_D = 128
_DOUT = 40
_NC = 2
_NS = 16
_NW = _NC * _NS
_CHUNK = 128
_SEG = 16
_R = 10112
_TPR = _R // _NS
_DUMMY = _N
_N0 = 128
_N1 = 32
_EPAD = _NS * (_N0 + _N1) * _CHUNK
_MS = max(_N0, _N1)


def _sc_agg(d, n0, n1):
    assert n0 % _SEG == 0 and n1 % _SEG == 0
    nseg0, nseg1 = n0 // _SEG, n1 // _SEG
    ms = max(n0, n1)
    nh = _SEG // 2
    mesh = plsc.VectorSubcoreMesh(core_axis_name="c", subcore_axis_name="s")

    @functools.partial(
        pl.kernel,
        mesh=mesh,
        out_type=jax.ShapeDtypeStruct((_NC, _R, d), jnp.float32),
        scratch_types=[
            pltpu.VMEM((_SEG, _CHUNK), jnp.int32),
            pltpu.VMEM((_SEG, _CHUNK), jnp.int32),
            pltpu.VMEM((_SEG, _CHUNK), jnp.int32),
            pltpu.VMEM((_SEG, _CHUNK), jnp.int32),
            pltpu.VMEM((_CHUNK, d), jnp.float32),
            pltpu.VMEM((_CHUNK, d), jnp.float32),
            pltpu.VMEM_SHARED((_R, d), jnp.float32),
            pltpu.SemaphoreType.DMA,
            pltpu.SemaphoreType.DMA,
            pltpu.SemaphoreType.DMA,
            pltpu.SemaphoreType.DMA,
            pltpu.SemaphoreType.DMA,
            pltpu.SemaphoreType.DMA,
        ],
    )
    def k(table_hbm, init1_hbm, src_hbm, dst_hbm, out_hbm,
          is0, is1, id0, id1, rows0, rows1, acc, t0, t1, g0, g1, s0, s1):
        c = lax.axis_index("c")
        s = lax.axis_index("s")
        base = pl.multiple_of((c * _NS + s) * ms, 8)
        r0 = s * _TPR

        @pl.when(c == 0)
        def _():
            pltpu.sync_copy(table_hbm.at[pl.ds(r0, _TPR)], acc.at[pl.ds(r0, _TPR)])

        @pl.when(c != 0)
        def _():
            pltpu.sync_copy(init1_hbm.at[pl.ds(r0, _TPR)], acc.at[pl.ds(r0, _TPR)])

        pltpu.sync_copy(src_hbm.at[pl.ds(base, _SEG)], is0)
        pltpu.sync_copy(dst_hbm.at[pl.ds(base, _SEG)], id0)
        plsc.subcore_barrier()

        ibufs = [(is0, id0, t0), (is1, id1, t1)]

        def segment(sb, db):
            pltpu.async_copy(table_hbm.at[sb.at[0]], rows0, g0)

            def body(jo, carry):
                j0 = 2 * jo
                j1 = j0 + 1
                pltpu.make_async_copy(table_hbm.at[sb.at[j0]], rows0, g0).wait()

                @pl.when(jo >= 1)
                def _():
                    pltpu.make_async_copy(rows1, acc.at[db.at[0]], s1).wait()

                pltpu.async_copy(table_hbm.at[sb.at[j1]], rows1, g1)
                pltpu.async_copy(rows0, acc.at[db.at[j0]], s0, add=True)
                pltpu.make_async_copy(table_hbm.at[sb.at[j1]], rows1, g1).wait()

                @pl.when(jo + 1 < nh)
                def _():
                    pltpu.make_async_copy(rows0, acc.at[db.at[0]], s0).wait()
                    pltpu.async_copy(table_hbm.at[sb.at[j0 + 2]], rows0, g0)

                pltpu.async_copy(rows1, acc.at[db.at[j1]], s1, add=True)
                return carry

            lax.fori_loop(0, nh, body, 0)
            pltpu.make_async_copy(rows0, acc.at[db.at[0]], s0).wait()
            pltpu.make_async_copy(rows1, acc.at[db.at[0]], s1).wait()

        def emit(nseg):
            for si in range(nseg):
                sb, db, _ = ibufs[si % 2]
                nsb, ndb, ntsem = ibufs[(si + 1) % 2]
                if si + 1 < nseg:
                    off = pl.multiple_of(base + (si + 1) * _SEG, 8)
                    pltpu.async_copy(src_hbm.at[pl.ds(off, _SEG)], nsb, ntsem)
                    pltpu.async_copy(dst_hbm.at[pl.ds(off, _SEG)], ndb, ntsem)
                segment(sb, db)
                if si + 1 < nseg:
                    pltpu.make_async_copy(src_hbm.at[pl.ds(base, _SEG)], nsb, ntsem).wait()
                    pltpu.make_async_copy(dst_hbm.at[pl.ds(base, _SEG)], ndb, ntsem).wait()

        @pl.when(c == 0)
        def _():
            emit(nseg0)

        @pl.when(c != 0)
        def _():
            emit(nseg1)

        plsc.subcore_barrier()
        pltpu.sync_copy(acc.at[pl.ds(r0, _TPR)], out_hbm.at[c, pl.ds(r0, _TPR)])

    return k


def _sc_deg():
    mesh = plsc.VectorSubcoreMesh(core_axis_name="c", subcore_axis_name="s")
    _EPW = _EPAD // _NW
    _NV = _EPW // 16

    @functools.partial(
        pl.kernel,
        mesh=mesh,
        out_type=jax.ShapeDtypeStruct((_NW, _R), jnp.float32),
        scratch_types=[
            pltpu.VMEM((_EPW,), jnp.int32),
            pltpu.VMEM((_R,), jnp.float32),
        ],
        compiler_params=pltpu.CompilerParams(needs_layout_passes=False),
    )
    def k(dst_hbm, out_hbm, idx_d, hist):
        c = lax.axis_index("c")
        s = lax.axis_index("s")
        wid = s * _NC + c
        pltpu.sync_copy(dst_hbm.at[pl.ds(wid * _EPW, _EPW)], idx_d)

        zeros16 = jnp.zeros((16,), jnp.float32)

        def zero_body(i, carry):
            hist[pl.ds(i * 16, 16)] = zeros16
            return carry

        lax.fori_loop(0, _R // 16, zero_body, 0)

        ones16 = jnp.full((16,), 1.0, jnp.float32)

        def body(i, carry):
            v = idx_d[pl.ds(i * 16, 16)]
            plsc.addupdate_scatter(hist, [v], ones16)
            return carry

        lax.fori_loop(0, _NV, body, 0)
        pltpu.sync_copy(hist, out_hbm.at[wid])

    return k


def _dinv_of(degp_ref):
    deg = jnp.sum(degp_ref[...], axis=0) + 1.0
    return lax.rsqrt(deg)


def _tc_prep_body(x_ref, w_ref, degp_ref, out_ref):
    dinv = _dinv_of(degp_ref)
    u = jnp.dot(x_ref[...], w_ref[...], preferred_element_type=jnp.float32)
    out_ref[...] = u * dinv[:, None]


def _tc_bn_body(p_ref, degp_ref, b_ref, g_ref, be_ref, w_ref, out_ref):
    dinv = _dinv_of(degp_ref)
    agg = p_ref[0] + p_ref[1]
    pre = agg * dinv[:, None] + b_ref[...]
    mask = lax.broadcasted_iota(jnp.int32, (_R, 1), 0) < _N
    mu = jnp.sum(jnp.where(mask, pre, 0.0), axis=0) / _N
    var = jnp.sum(jnp.where(mask, (pre - mu) ** 2, 0.0), axis=0) / _N
    y = (pre - mu) * lax.rsqrt(var + 1e-5) * g_ref[...] + be_ref[...]
    r = jnp.maximum(y, 0.0)
    u = jnp.dot(r, w_ref[...], preferred_element_type=jnp.float32)
    out_ref[...] = jnp.where(mask, u * dinv[:, None], 0.0)


def _tc_out_body(p_ref, degp_ref, b_ref, out_ref):
    dinv = _dinv_of(degp_ref)
    agg = p_ref[0] + p_ref[1]
    o = agg[:_N, :_DOUT] * dinv[:_N, None] + b_ref[...]
    m = jnp.max(o, axis=1, keepdims=True)
    o = o - m
    out_ref[...] = o - jnp.log(jnp.sum(jnp.exp(o), axis=1, keepdims=True))


def kernel(x, adj_t, W1, b1, g1, be1, W2, b2, g2, be2, W3, b3):
    def _slabs(idx):
        ep = jnp.concatenate([idx, jnp.full((_EPAD - _E,), _DUMMY, jnp.int32)])
        e0 = _NS * _N0 * _CHUNK
        p0 = ep[:e0].reshape(_NS, _N0, _CHUNK)
        p1 = ep[e0:].reshape(_NS, _N1, _CHUNK)
        f0 = jnp.full((_NS, _MS - _N0, _CHUNK), _DUMMY, jnp.int32)
        f1 = jnp.full((_NS, _MS - _N1, _CHUNK), _DUMMY, jnp.int32)
        slabs = jnp.concatenate([jnp.concatenate([p0, f0], 1),
                                 jnp.concatenate([p1, f1], 1)])
        return ep, slabs.reshape(_NW * _MS, _CHUNK)

    _, srcp = _slabs(adj_t[0])
    dstp_flat, dstp = _slabs(adj_t[1])

    degp = _sc_deg()(dstp_flat)

    xpad = jnp.concatenate([x, jnp.zeros((_R - _N, _D), jnp.float32)])
    h1 = pl.pallas_call(
        _tc_prep_body,
        out_shape=jax.ShapeDtypeStruct((_R, _D), jnp.float32),
    )(xpad, W1, degp)

    z128 = jnp.zeros((_R, _D), jnp.float32)
    p1 = _sc_agg(_D, _N0, _N1)(h1, z128, srcp, dstp)
    h2 = pl.pallas_call(
        _tc_bn_body,
        out_shape=jax.ShapeDtypeStruct((_R, _D), jnp.float32),
    )(p1, degp, b1, g1, be1, W2)

    p2 = _sc_agg(_D, _N0, _N1)(h2, z128, srcp, dstp)
    W3p = jnp.concatenate([W3, jnp.zeros((_D, _D - _DOUT), jnp.float32)], axis=1)
    h3 = pl.pallas_call(
        _tc_bn_body,
        out_shape=jax.ShapeDtypeStruct((_R, _D), jnp.float32),
    )(p2, degp, b2, g2, be2, W3p)

    p3 = _sc_agg(_D, _N0, _N1)(h3, z128, srcp, dstp)
    out = pl.pallas_call(
        _tc_out_body,
        out_shape=jax.ShapeDtypeStruct((_N, _DOUT), jnp.float32),
    )(p3, degp, b3)
    return out


# speedup vs baseline: 3.4016x; 3.4016x over previous
import functools

import jax
import jax.numpy as jnp
from jax import lax
from jax.experimental import pallas as pl
from jax.experimental.pallas import tpu as pltpu
from jax.experimental.pallas import tpu_sc as plsc

_N = 10000
_E = 320000
_D = 128
_DOUT = 40
_NC = 2
_NS = 16
_NW = _NC * _NS
_CHUNK = 128
_SEG = 16
_R = 10112
_TPR = _R // _NS
_DUMMY = _N
_N0 = 80
_N1 = 80
_EPAD = _NS * (_N0 + _N1) * _CHUNK
_MS = max(_N0, _N1)


def _sc_agg(d, n0, n1):
    assert n0 % _SEG == 0 and n1 % _SEG == 0
    nseg0, nseg1 = n0 // _SEG, n1 // _SEG
    ms = max(n0, n1)
    nh = _SEG // 2
    mesh = plsc.VectorSubcoreMesh(core_axis_name="c", subcore_axis_name="s")

    @functools.partial(
        pl.kernel,
        mesh=mesh,
        out_type=jax.ShapeDtypeStruct((_NC, _R, d), jnp.float32),
        scratch_types=[
            pltpu.VMEM((_SEG, _CHUNK), jnp.int32),
            pltpu.VMEM((_SEG, _CHUNK), jnp.int32),
            pltpu.VMEM((_SEG, _CHUNK), jnp.int32),
            pltpu.VMEM((_SEG, _CHUNK), jnp.int32),
            pltpu.VMEM((_CHUNK, d), jnp.float32),
            pltpu.VMEM((_CHUNK, d), jnp.float32),
            pltpu.VMEM_SHARED((_R, d), jnp.float32),
            pltpu.SemaphoreType.DMA,
            pltpu.SemaphoreType.DMA,
            pltpu.SemaphoreType.DMA,
            pltpu.SemaphoreType.DMA,
            pltpu.SemaphoreType.DMA,
            pltpu.SemaphoreType.DMA,
        ],
    )
    def k(table_hbm, init1_hbm, src_hbm, dst_hbm, out_hbm,
          is0, is1, id0, id1, rows0, rows1, acc, t0, t1, g0, g1, s0, s1):
        c = lax.axis_index("c")
        s = lax.axis_index("s")
        base = pl.multiple_of((c * _NS + s) * ms, 8)
        r0 = s * _TPR

        @pl.when(c == 0)
        def _():
            pltpu.sync_copy(table_hbm.at[pl.ds(r0, _TPR)], acc.at[pl.ds(r0, _TPR)])

        @pl.when(c != 0)
        def _():
            pltpu.sync_copy(init1_hbm.at[pl.ds(r0, _TPR)], acc.at[pl.ds(r0, _TPR)])

        pltpu.sync_copy(src_hbm.at[pl.ds(base, _SEG)], is0)
        pltpu.sync_copy(dst_hbm.at[pl.ds(base, _SEG)], id0)
        plsc.subcore_barrier()

        ibufs = [(is0, id0, t0), (is1, id1, t1)]

        def segment(sb, db):
            pltpu.async_copy(table_hbm.at[sb.at[0]], rows0, g0)

            def body(jo, carry):
                j0 = 2 * jo
                j1 = j0 + 1
                pltpu.make_async_copy(table_hbm.at[sb.at[j0]], rows0, g0).wait()

                @pl.when(jo >= 1)
                def _():
                    pltpu.make_async_copy(rows1, acc.at[db.at[0]], s1).wait()

                pltpu.async_copy(table_hbm.at[sb.at[j1]], rows1, g1)
                pltpu.async_copy(rows0, acc.at[db.at[j0]], s0, add=True)
                pltpu.make_async_copy(table_hbm.at[sb.at[j1]], rows1, g1).wait()

                @pl.when(jo + 1 < nh)
                def _():
                    pltpu.make_async_copy(rows0, acc.at[db.at[0]], s0).wait()
                    pltpu.async_copy(table_hbm.at[sb.at[j0 + 2]], rows0, g0)

                pltpu.async_copy(rows1, acc.at[db.at[j1]], s1, add=True)
                return carry

            lax.fori_loop(0, nh, body, 0)
            pltpu.make_async_copy(rows0, acc.at[db.at[0]], s0).wait()
            pltpu.make_async_copy(rows1, acc.at[db.at[0]], s1).wait()

        def emit(nseg):
            for si in range(nseg):
                sb, db, _ = ibufs[si % 2]
                nsb, ndb, ntsem = ibufs[(si + 1) % 2]
                if si + 1 < nseg:
                    off = pl.multiple_of(base + (si + 1) * _SEG, 8)
                    pltpu.async_copy(src_hbm.at[pl.ds(off, _SEG)], nsb, ntsem)
                    pltpu.async_copy(dst_hbm.at[pl.ds(off, _SEG)], ndb, ntsem)
                segment(sb, db)
                if si + 1 < nseg:
                    pltpu.make_async_copy(src_hbm.at[pl.ds(base, _SEG)], nsb, ntsem).wait()
                    pltpu.make_async_copy(dst_hbm.at[pl.ds(base, _SEG)], ndb, ntsem).wait()

        @pl.when(c == 0)
        def _():
            emit(nseg0)

        @pl.when(c != 0)
        def _():
            emit(nseg1)

        plsc.subcore_barrier()
        pltpu.sync_copy(acc.at[pl.ds(r0, _TPR)], out_hbm.at[c, pl.ds(r0, _TPR)])

    return k


def _sc_deg():
    mesh = plsc.VectorSubcoreMesh(core_axis_name="c", subcore_axis_name="s")
    _EPW = _EPAD // _NW
    _NV = _EPW // 16

    @functools.partial(
        pl.kernel,
        mesh=mesh,
        out_type=jax.ShapeDtypeStruct((_NW, _R), jnp.float32),
        scratch_types=[
            pltpu.VMEM((_EPW,), jnp.int32),
            pltpu.VMEM((_R,), jnp.float32),
        ],
        compiler_params=pltpu.CompilerParams(needs_layout_passes=False),
    )
    def k(dst_hbm, out_hbm, idx_d, hist):
        c = lax.axis_index("c")
        s = lax.axis_index("s")
        wid = s * _NC + c
        pltpu.sync_copy(dst_hbm.at[pl.ds(wid * _EPW, _EPW)], idx_d)

        zeros16 = jnp.zeros((16,), jnp.float32)

        def zero_body(i, carry):
            hist[pl.ds(i * 16, 16)] = zeros16
            return carry

        lax.fori_loop(0, _R // 16, zero_body, 0)

        ones16 = jnp.full((16,), 1.0, jnp.float32)

        def body(i, carry):
            v = idx_d[pl.ds(i * 16, 16)]
            plsc.addupdate_scatter(hist, [v], ones16)
            return carry

        lax.fori_loop(0, _NV, body, 0)
        pltpu.sync_copy(hist, out_hbm.at[wid])

    return k


def _dinv_of(degp_ref):
    deg = jnp.sum(degp_ref[...], axis=0) + 1.0
    return lax.rsqrt(deg)


def _tc_prep_body(x_ref, w_ref, degp_ref, out_ref):
    dinv = _dinv_of(degp_ref)
    u = jnp.dot(x_ref[...], w_ref[...], preferred_element_type=jnp.float32)
    out_ref[...] = u * dinv[:, None]


def _tc_bn_body(p_ref, degp_ref, b_ref, g_ref, be_ref, w_ref, out_ref):
    dinv = _dinv_of(degp_ref)
    agg = p_ref[0] + p_ref[1]
    pre = agg * dinv[:, None] + b_ref[...]
    mask = lax.broadcasted_iota(jnp.int32, (_R, 1), 0) < _N
    mu = jnp.sum(jnp.where(mask, pre, 0.0), axis=0) / _N
    var = jnp.sum(jnp.where(mask, (pre - mu) ** 2, 0.0), axis=0) / _N
    y = (pre - mu) * lax.rsqrt(var + 1e-5) * g_ref[...] + be_ref[...]
    r = jnp.maximum(y, 0.0)
    u = jnp.dot(r, w_ref[...], preferred_element_type=jnp.float32)
    out_ref[...] = jnp.where(mask, u * dinv[:, None], 0.0)


def _tc_out_body(p_ref, degp_ref, b_ref, out_ref):
    dinv = _dinv_of(degp_ref)
    agg = p_ref[0] + p_ref[1]
    o = agg[:_N, :_DOUT] * dinv[:_N, None] + b_ref[...]
    m = jnp.max(o, axis=1, keepdims=True)
    o = o - m
    out_ref[...] = o - jnp.log(jnp.sum(jnp.exp(o), axis=1, keepdims=True))


def kernel(x, adj_t, W1, b1, g1, be1, W2, b2, g2, be2, W3, b3):
    _dummy_fill = _DUMMY + (jnp.arange(_EPAD - _E, dtype=jnp.int32) % (_R - _N))

    def _slabs(idx):
        ep = jnp.concatenate([idx, _dummy_fill])
        e0 = _NS * _N0 * _CHUNK
        p0 = ep[:e0].reshape(_NS, _N0, _CHUNK)
        p1 = ep[e0:].reshape(_NS, _N1, _CHUNK)
        f0 = jnp.full((_NS, _MS - _N0, _CHUNK), _DUMMY, jnp.int32)
        f1 = jnp.full((_NS, _MS - _N1, _CHUNK), _DUMMY, jnp.int32)
        slabs = jnp.concatenate([jnp.concatenate([p0, f0], 1),
                                 jnp.concatenate([p1, f1], 1)])
        return ep, slabs.reshape(_NW * _MS, _CHUNK)

    _, srcp = _slabs(adj_t[0])
    dstp_flat, dstp = _slabs(adj_t[1])

    degp = _sc_deg()(dstp_flat)

    xpad = jnp.concatenate([x, jnp.zeros((_R - _N, _D), jnp.float32)])
    h1 = pl.pallas_call(
        _tc_prep_body,
        out_shape=jax.ShapeDtypeStruct((_R, _D), jnp.float32),
    )(xpad, W1, degp)

    z128 = jnp.zeros((_R, _D), jnp.float32)
    p1 = _sc_agg(_D, _N0, _N1)(h1, z128, srcp, dstp)
    h2 = pl.pallas_call(
        _tc_bn_body,
        out_shape=jax.ShapeDtypeStruct((_R, _D), jnp.float32),
    )(p1, degp, b1, g1, be1, W2)

    p2 = _sc_agg(_D, _N0, _N1)(h2, z128, srcp, dstp)
    W3p = jnp.concatenate([W3, jnp.zeros((_D, _D - _DOUT), jnp.float32)], axis=1)
    h3 = pl.pallas_call(
        _tc_bn_body,
        out_shape=jax.ShapeDtypeStruct((_R, _D), jnp.float32),
    )(p2, degp, b2, g2, be2, W3p)

    p3 = _sc_agg(_D, _N0, _N1)(h3, z128, srcp, dstp)
    out = pl.pallas_call(
        _tc_out_body,
        out_shape=jax.ShapeDtypeStruct((_N, _DOUT), jnp.float32),
    )(p3, degp, b3)
    return out


# speedup vs baseline: 3.5121x; 1.0325x over previous
import functools

import jax
import jax.numpy as jnp
from jax import lax
from jax.experimental import pallas as pl
from jax.experimental.pallas import tpu as pltpu
from jax.experimental.pallas import tpu_sc as plsc

_N = 10000
_E = 320000
_D = 128
_DOUT = 40
_NC = 2
_NS = 16
_NW = _NC * _NS
_CHUNK = 128
_SEG = 16
_R = 10112
_TPR = _R // _NS
_DUMMY = _N
_N0 = 80
_N1 = 80
_EPAD = _NS * (_N0 + _N1) * _CHUNK
_MS = max(_N0, _N1)


def _sc_agg(d, n0, n1):
    assert n0 % _SEG == 0 and n1 % _SEG == 0
    nseg0, nseg1 = n0 // _SEG, n1 // _SEG
    ms = max(n0, n1)
    nh = _SEG // 2
    mesh = plsc.VectorSubcoreMesh(core_axis_name="c", subcore_axis_name="s")

    @functools.partial(
        pl.kernel,
        mesh=mesh,
        out_type=jax.ShapeDtypeStruct((_NC, _R, d), jnp.float32),
        scratch_types=[
            pltpu.VMEM((_SEG, _CHUNK), jnp.int32),
            pltpu.VMEM((_SEG, _CHUNK), jnp.int32),
            pltpu.VMEM((_SEG, _CHUNK), jnp.int32),
            pltpu.VMEM((_SEG, _CHUNK), jnp.int32),
            pltpu.VMEM((_CHUNK, d), jnp.float32),
            pltpu.VMEM((_CHUNK, d), jnp.float32),
            pltpu.VMEM_SHARED((_R, d), jnp.float32),
            pltpu.SemaphoreType.DMA,
            pltpu.SemaphoreType.DMA,
            pltpu.SemaphoreType.DMA,
            pltpu.SemaphoreType.DMA,
            pltpu.SemaphoreType.DMA,
            pltpu.SemaphoreType.DMA,
            pltpu.SemaphoreType.DMA,
        ],
    )
    def k(table_hbm, init1_hbm, src_hbm, dst_hbm, out_hbm,
          is0, is1, id0, id1, rows0, rows1, acc, ti, t0, t1, g0, g1, s0, s1):
        c = lax.axis_index("c")
        s = lax.axis_index("s")
        base = pl.multiple_of((c * _NS + s) * ms, 8)
        r0 = s * _TPR

        @pl.when(c == 0)
        def _():
            pltpu.async_copy(table_hbm.at[pl.ds(r0, _TPR)], acc.at[pl.ds(r0, _TPR)], ti)

        @pl.when(c != 0)
        def _():
            pltpu.async_copy(init1_hbm.at[pl.ds(r0, _TPR)], acc.at[pl.ds(r0, _TPR)], ti)

        pltpu.sync_copy(src_hbm.at[pl.ds(base, _SEG)], is0)
        pltpu.sync_copy(dst_hbm.at[pl.ds(base, _SEG)], id0)
        pltpu.async_copy(table_hbm.at[is0.at[0]], rows0, g0)
        pltpu.make_async_copy(table_hbm.at[pl.ds(r0, _TPR)], acc.at[pl.ds(r0, _TPR)], ti).wait()
        plsc.subcore_barrier()

        ibufs = [(is0, id0, t0), (is1, id1, t1)]

        def emit(nseg):
            for si in range(nseg):
                sb, db, _ = ibufs[si % 2]
                last = si + 1 == nseg
                if si > 0:
                    pltpu.make_async_copy(rows1, acc.at[db.at[0]], s1).wait()
                if not last:
                    nsb, ndb, ntsem = ibufs[(si + 1) % 2]
                    off = pl.multiple_of(base + (si + 1) * _SEG, 8)
                    pltpu.async_copy(src_hbm.at[pl.ds(off, _SEG)], nsb, ntsem)
                    pltpu.async_copy(dst_hbm.at[pl.ds(off, _SEG)], ndb, ntsem)

                def body(jo, carry, sb=sb, db=db, si=si, last=last):
                    j0 = 2 * jo
                    j1 = j0 + 1
                    pltpu.make_async_copy(table_hbm.at[sb.at[j0]], rows0, g0).wait()

                    @pl.when(jo >= 1)
                    def _():
                        pltpu.make_async_copy(rows1, acc.at[db.at[0]], s1).wait()

                    pltpu.async_copy(table_hbm.at[sb.at[j1]], rows1, g1)
                    pltpu.async_copy(rows0, acc.at[db.at[j0]], s0, add=True)
                    pltpu.make_async_copy(table_hbm.at[sb.at[j1]], rows1, g1).wait()

                    @pl.when(jo + 1 < nh)
                    def _():
                        pltpu.make_async_copy(rows0, acc.at[db.at[0]], s0).wait()
                        pltpu.async_copy(table_hbm.at[sb.at[j0 + 2]], rows0, g0)

                    if not last:
                        nsb2, _, ntsem2 = ibufs[(si + 1) % 2]

                        @pl.when(jo + 1 == nh)
                        def _():
                            pltpu.make_async_copy(rows0, acc.at[db.at[0]], s0).wait()
                            pltpu.make_async_copy(
                                src_hbm.at[pl.ds(base, _SEG)], nsb2, ntsem2).wait()
                            pltpu.make_async_copy(
                                src_hbm.at[pl.ds(base, _SEG)], nsb2, ntsem2).wait()
                            pltpu.async_copy(table_hbm.at[nsb2.at[0]], rows0, g0)

                    pltpu.async_copy(rows1, acc.at[db.at[j1]], s1, add=True)
                    return carry

                lax.fori_loop(0, nh, body, 0)

            pltpu.make_async_copy(rows0, acc.at[ibufs[0][1].at[0]], s0).wait()
            pltpu.make_async_copy(rows1, acc.at[ibufs[0][1].at[0]], s1).wait()

        @pl.when(c == 0)
        def _():
            emit(nseg0)

        @pl.when(c != 0)
        def _():
            emit(nseg1)

        plsc.subcore_barrier()
        pltpu.sync_copy(acc.at[pl.ds(r0, _TPR)], out_hbm.at[c, pl.ds(r0, _TPR)])

    return k


def _sc_deg():
    mesh = plsc.VectorSubcoreMesh(core_axis_name="c", subcore_axis_name="s")
    _EPW = _EPAD // _NW
    _NV = _EPW // 16

    @functools.partial(
        pl.kernel,
        mesh=mesh,
        out_type=jax.ShapeDtypeStruct((_NW, _R), jnp.float32),
        scratch_types=[
            pltpu.VMEM((_EPW,), jnp.int32),
            pltpu.VMEM((_R,), jnp.float32),
        ],
        compiler_params=pltpu.CompilerParams(needs_layout_passes=False),
    )
    def k(dst_hbm, out_hbm, idx_d, hist):
        c = lax.axis_index("c")
        s = lax.axis_index("s")
        wid = s * _NC + c
        pltpu.sync_copy(dst_hbm.at[pl.ds(wid * _EPW, _EPW)], idx_d)

        zeros16 = jnp.zeros((16,), jnp.float32)

        def zero_body(i, carry):
            hist[pl.ds(i * 16, 16)] = zeros16
            return carry

        lax.fori_loop(0, _R // 16, zero_body, 0)

        ones16 = jnp.full((16,), 1.0, jnp.float32)

        def body(i, carry):
            v = idx_d[pl.ds(i * 16, 16)]
            plsc.addupdate_scatter(hist, [v], ones16)
            return carry

        lax.fori_loop(0, _NV, body, 0)
        pltpu.sync_copy(hist, out_hbm.at[wid])

    return k


def _dinv_of(degp_ref):
    deg = jnp.sum(degp_ref[...], axis=0) + 1.0
    return lax.rsqrt(deg)


def _tc_prep_body(x_ref, w_ref, degp_ref, out_ref):
    dinv = _dinv_of(degp_ref)
    u = jnp.dot(x_ref[...], w_ref[...], preferred_element_type=jnp.float32)
    out_ref[...] = u * dinv[:, None]


def _tc_bn_body(p_ref, degp_ref, b_ref, g_ref, be_ref, w_ref, out_ref):
    dinv = _dinv_of(degp_ref)
    agg = p_ref[0] + p_ref[1]
    pre = agg * dinv[:, None] + b_ref[...]
    mask = lax.broadcasted_iota(jnp.int32, (_R, 1), 0) < _N
    mu = jnp.sum(jnp.where(mask, pre, 0.0), axis=0) / _N
    var = jnp.sum(jnp.where(mask, (pre - mu) ** 2, 0.0), axis=0) / _N
    y = (pre - mu) * lax.rsqrt(var + 1e-5) * g_ref[...] + be_ref[...]
    r = jnp.maximum(y, 0.0)
    u = jnp.dot(r, w_ref[...], preferred_element_type=jnp.float32)
    out_ref[...] = jnp.where(mask, u * dinv[:, None], 0.0)


def _tc_out_body(p_ref, degp_ref, b_ref, out_ref):
    dinv = _dinv_of(degp_ref)
    agg = p_ref[0] + p_ref[1]
    o = agg[:_N, :_DOUT] * dinv[:_N, None] + b_ref[...]
    m = jnp.max(o, axis=1, keepdims=True)
    o = o - m
    out_ref[...] = o - jnp.log(jnp.sum(jnp.exp(o), axis=1, keepdims=True))


def kernel(x, adj_t, W1, b1, g1, be1, W2, b2, g2, be2, W3, b3):
    _dummy_fill = _DUMMY + (jnp.arange(_EPAD - _E, dtype=jnp.int32) % (_R - _N))

    def _slabs(idx):
        ep = jnp.concatenate([idx, _dummy_fill])
        e0 = _NS * _N0 * _CHUNK
        p0 = ep[:e0].reshape(_NS, _N0, _CHUNK)
        p1 = ep[e0:].reshape(_NS, _N1, _CHUNK)
        f0 = jnp.full((_NS, _MS - _N0, _CHUNK), _DUMMY, jnp.int32)
        f1 = jnp.full((_NS, _MS - _N1, _CHUNK), _DUMMY, jnp.int32)
        slabs = jnp.concatenate([jnp.concatenate([p0, f0], 1),
                                 jnp.concatenate([p1, f1], 1)])
        return ep, slabs.reshape(_NW * _MS, _CHUNK)

    _, srcp = _slabs(adj_t[0])
    dstp_flat, dstp = _slabs(adj_t[1])

    degp = _sc_deg()(dstp_flat)

    xpad = jnp.concatenate([x, jnp.zeros((_R - _N, _D), jnp.float32)])
    h1 = pl.pallas_call(
        _tc_prep_body,
        out_shape=jax.ShapeDtypeStruct((_R, _D), jnp.float32),
    )(xpad, W1, degp)

    z128 = jnp.zeros((_R, _D), jnp.float32)
    p1 = _sc_agg(_D, _N0, _N1)(h1, z128, srcp, dstp)
    h2 = pl.pallas_call(
        _tc_bn_body,
        out_shape=jax.ShapeDtypeStruct((_R, _D), jnp.float32),
    )(p1, degp, b1, g1, be1, W2)

    p2 = _sc_agg(_D, _N0, _N1)(h2, z128, srcp, dstp)
    W3p = jnp.concatenate([W3, jnp.zeros((_D, _D - _DOUT), jnp.float32)], axis=1)
    h3 = pl.pallas_call(
        _tc_bn_body,
        out_shape=jax.ShapeDtypeStruct((_R, _D), jnp.float32),
    )(p2, degp, b2, g2, be2, W3p)

    p3 = _sc_agg(_D, _N0, _N1)(h3, z128, srcp, dstp)
    out = pl.pallas_call(
        _tc_out_body,
        out_shape=jax.ShapeDtypeStruct((_N, _DOUT), jnp.float32),
    )(p3, degp, b3)
    return out


# speedup vs baseline: 4.1333x; 1.1769x over previous
import functools

import jax
import jax.numpy as jnp
from jax import lax
from jax.experimental import pallas as pl
from jax.experimental.pallas import tpu as pltpu
from jax.experimental.pallas import tpu_sc as plsc

_N = 10000
_E = 320000
_D = 128
_DOUT = 40
_NC = 2
_NS = 16
_NW = _NC * _NS
_CHUNK = 128
_SEG = 16
_R = 10112
_TPR = _R // _NS
_DUMMY = _N
_N0 = 80
_N1 = 80
_EPAD = _NS * (_N0 + _N1) * _CHUNK
_MS = max(_N0, _N1)


def _sc_agg(d, n0, n1):
    assert n0 % _SEG == 0 and n1 % _SEG == 0
    nseg0, nseg1 = n0 // _SEG, n1 // _SEG
    ms = max(n0, n1)
    nh = _SEG // 2
    mesh = plsc.VectorSubcoreMesh(core_axis_name="c", subcore_axis_name="s")

    @functools.partial(
        pl.kernel,
        mesh=mesh,
        out_type=jax.ShapeDtypeStruct((_NC, _R, d), jnp.float32),
        scratch_types=[
            pltpu.VMEM((_SEG, _CHUNK), jnp.int32),
            pltpu.VMEM((_SEG, _CHUNK), jnp.int32),
            pltpu.VMEM((_SEG, _CHUNK), jnp.int32),
            pltpu.VMEM((_SEG, _CHUNK), jnp.int32),
            pltpu.VMEM((_CHUNK, d), jnp.float32),
            pltpu.VMEM((_CHUNK, d), jnp.float32),
            pltpu.VMEM_SHARED((_R, d), jnp.float32),
            pltpu.SemaphoreType.DMA,
            pltpu.SemaphoreType.DMA,
            pltpu.SemaphoreType.DMA,
            pltpu.SemaphoreType.DMA,
            pltpu.SemaphoreType.DMA,
            pltpu.SemaphoreType.DMA,
            pltpu.SemaphoreType.DMA,
        ],
    )
    def k(table_hbm, init1_hbm, src_hbm, dst_hbm, out_hbm,
          is0, is1, id0, id1, rows0, rows1, acc, ti, t0, t1, g0, g1, s0, s1):
        c = lax.axis_index("c")
        s = lax.axis_index("s")
        base = pl.multiple_of((c * _NS + s) * ms, 8)
        r0 = s * _TPR

        @pl.when(c == 0)
        def _():
            pltpu.async_copy(table_hbm.at[pl.ds(r0, _TPR)], acc.at[pl.ds(r0, _TPR)], ti)

        @pl.when(c != 0)
        def _():
            pltpu.async_copy(init1_hbm.at[pl.ds(r0, _TPR)], acc.at[pl.ds(r0, _TPR)], ti)

        pltpu.sync_copy(src_hbm.at[pl.ds(base, _SEG)], is0)
        pltpu.sync_copy(dst_hbm.at[pl.ds(base, _SEG)], id0)
        pltpu.async_copy(table_hbm.at[is0.at[0]], rows0, g0)
        pltpu.make_async_copy(table_hbm.at[pl.ds(r0, _TPR)], acc.at[pl.ds(r0, _TPR)], ti).wait()
        plsc.subcore_barrier()

        ibufs = [(is0, id0, t0), (is1, id1, t1)]

        def emit(nseg):
            for si in range(nseg):
                sb, db, _ = ibufs[si % 2]
                last = si + 1 == nseg
                if si > 0:
                    pltpu.make_async_copy(rows1, acc.at[db.at[0]], s1).wait()
                if not last:
                    nsb, ndb, ntsem = ibufs[(si + 1) % 2]
                    off = pl.multiple_of(base + (si + 1) * _SEG, 8)
                    pltpu.async_copy(src_hbm.at[pl.ds(off, _SEG)], nsb, ntsem)
                    pltpu.async_copy(dst_hbm.at[pl.ds(off, _SEG)], ndb, ntsem)

                def body(jo, carry, sb=sb, db=db, si=si, last=last):
                    j0 = 2 * jo
                    j1 = j0 + 1

                    @pl.when(jo >= 1)
                    def _():
                        pltpu.make_async_copy(rows1, acc.at[db.at[0]], s1).wait()

                    pltpu.async_copy(table_hbm.at[sb.at[j1]], rows1, g1)
                    pltpu.make_async_copy(table_hbm.at[sb.at[j0]], rows0, g0).wait()
                    pltpu.async_copy(rows0, acc.at[db.at[j0]], s0, add=True)
                    pltpu.make_async_copy(rows0, acc.at[db.at[0]], s0).wait()

                    @pl.when(jo + 1 < nh)
                    def _():
                        pltpu.async_copy(table_hbm.at[sb.at[j0 + 2]], rows0, g0)

                    if not last:
                        nsb2, _, ntsem2 = ibufs[(si + 1) % 2]

                        @pl.when(jo + 1 == nh)
                        def _():
                            pltpu.make_async_copy(
                                src_hbm.at[pl.ds(base, _SEG)], nsb2, ntsem2).wait()
                            pltpu.make_async_copy(
                                src_hbm.at[pl.ds(base, _SEG)], nsb2, ntsem2).wait()
                            pltpu.async_copy(table_hbm.at[nsb2.at[0]], rows0, g0)

                    pltpu.make_async_copy(table_hbm.at[sb.at[j1]], rows1, g1).wait()
                    pltpu.async_copy(rows1, acc.at[db.at[j1]], s1, add=True)
                    return carry

                lax.fori_loop(0, nh, body, 0)

            pltpu.make_async_copy(rows1, acc.at[ibufs[0][1].at[0]], s1).wait()

        @pl.when(c == 0)
        def _():
            emit(nseg0)

        @pl.when(c != 0)
        def _():
            emit(nseg1)

        plsc.subcore_barrier()
        pltpu.sync_copy(acc.at[pl.ds(r0, _TPR)], out_hbm.at[c, pl.ds(r0, _TPR)])

    return k


def _sc_deg():
    mesh = plsc.VectorSubcoreMesh(core_axis_name="c", subcore_axis_name="s")
    _EPW = _EPAD // _NW
    _NV = _EPW // 16

    @functools.partial(
        pl.kernel,
        mesh=mesh,
        out_type=jax.ShapeDtypeStruct((_NW, _R), jnp.float32),
        scratch_types=[
            pltpu.VMEM((_EPW,), jnp.int32),
            pltpu.VMEM((_R,), jnp.float32),
        ],
        compiler_params=pltpu.CompilerParams(needs_layout_passes=False),
    )
    def k(dst_hbm, out_hbm, idx_d, hist):
        c = lax.axis_index("c")
        s = lax.axis_index("s")
        wid = s * _NC + c
        pltpu.sync_copy(dst_hbm.at[pl.ds(wid * _EPW, _EPW)], idx_d)

        zeros16 = jnp.zeros((16,), jnp.float32)

        def zero_body(i, carry):
            hist[pl.ds(i * 16, 16)] = zeros16
            return carry

        lax.fori_loop(0, _R // 16, zero_body, 0)

        ones16 = jnp.full((16,), 1.0, jnp.float32)

        def body(i, carry):
            v = idx_d[pl.ds(i * 16, 16)]
            plsc.addupdate_scatter(hist, [v], ones16)
            return carry

        lax.fori_loop(0, _NV, body, 0)
        pltpu.sync_copy(hist, out_hbm.at[wid])

    return k


def _dinv_of(degp_ref):
    deg = jnp.sum(degp_ref[...], axis=0) + 1.0
    return lax.rsqrt(deg)


def _tc_prep_body(x_ref, w_ref, degp_ref, out_ref):
    dinv = _dinv_of(degp_ref)
    u = jnp.dot(x_ref[...], w_ref[...], preferred_element_type=jnp.float32)
    out_ref[...] = u * dinv[:, None]


def _tc_bn_body(p_ref, degp_ref, b_ref, g_ref, be_ref, w_ref, out_ref):
    dinv = _dinv_of(degp_ref)
    agg = p_ref[0] + p_ref[1]
    pre = agg * dinv[:, None] + b_ref[...]
    mask = lax.broadcasted_iota(jnp.int32, (_R, 1), 0) < _N
    mu = jnp.sum(jnp.where(mask, pre, 0.0), axis=0) / _N
    var = jnp.sum(jnp.where(mask, (pre - mu) ** 2, 0.0), axis=0) / _N
    y = (pre - mu) * lax.rsqrt(var + 1e-5) * g_ref[...] + be_ref[...]
    r = jnp.maximum(y, 0.0)
    u = jnp.dot(r, w_ref[...], preferred_element_type=jnp.float32)
    out_ref[...] = jnp.where(mask, u * dinv[:, None], 0.0)


def _tc_out_body(p_ref, degp_ref, b_ref, out_ref):
    dinv = _dinv_of(degp_ref)
    agg = p_ref[0] + p_ref[1]
    o = agg[:_N, :_DOUT] * dinv[:_N, None] + b_ref[...]
    m = jnp.max(o, axis=1, keepdims=True)
    o = o - m
    out_ref[...] = o - jnp.log(jnp.sum(jnp.exp(o), axis=1, keepdims=True))


def kernel(x, adj_t, W1, b1, g1, be1, W2, b2, g2, be2, W3, b3):
    _dummy_fill = _DUMMY + (jnp.arange(_EPAD - _E, dtype=jnp.int32) % (_R - _N))

    def _slabs(idx):
        ep = jnp.concatenate([idx, _dummy_fill])
        e0 = _NS * _N0 * _CHUNK
        p0 = ep[:e0].reshape(_NS, _N0, _CHUNK)
        p1 = ep[e0:].reshape(_NS, _N1, _CHUNK)
        f0 = jnp.full((_NS, _MS - _N0, _CHUNK), _DUMMY, jnp.int32)
        f1 = jnp.full((_NS, _MS - _N1, _CHUNK), _DUMMY, jnp.int32)
        slabs = jnp.concatenate([jnp.concatenate([p0, f0], 1),
                                 jnp.concatenate([p1, f1], 1)])
        return ep, slabs.reshape(_NW * _MS, _CHUNK)

    _, srcp = _slabs(adj_t[0])
    dstp_flat, dstp = _slabs(adj_t[1])

    degp = _sc_deg()(dstp_flat)

    xpad = jnp.concatenate([x, jnp.zeros((_R - _N, _D), jnp.float32)])
    h1 = pl.pallas_call(
        _tc_prep_body,
        out_shape=jax.ShapeDtypeStruct((_R, _D), jnp.float32),
    )(xpad, W1, degp)

    z128 = jnp.zeros((_R, _D), jnp.float32)
    p1 = _sc_agg(_D, _N0, _N1)(h1, z128, srcp, dstp)
    h2 = pl.pallas_call(
        _tc_bn_body,
        out_shape=jax.ShapeDtypeStruct((_R, _D), jnp.float32),
    )(p1, degp, b1, g1, be1, W2)

    p2 = _sc_agg(_D, _N0, _N1)(h2, z128, srcp, dstp)
    W3p = jnp.concatenate([W3, jnp.zeros((_D, _D - _DOUT), jnp.float32)], axis=1)
    h3 = pl.pallas_call(
        _tc_bn_body,
        out_shape=jax.ShapeDtypeStruct((_R, _D), jnp.float32),
    )(p2, degp, b2, g2, be2, W3p)

    p3 = _sc_agg(_D, _N0, _N1)(h3, z128, srcp, dstp)
    out = pl.pallas_call(
        _tc_out_body,
        out_shape=jax.ShapeDtypeStruct((_N, _DOUT), jnp.float32),
    )(p3, degp, b3)
    return out
